# baseline (speedup 1.0000x reference)
import jax
import jax.numpy as jnp
from jax import lax
from jax.experimental import pallas as pl
from jax.experimental.pallas import tpu as pltpu
from jax.experimental.pallas import tpu_sc as plsc

_TEMPERATURE = 0.07
_BASE_TEMPERATURE = 0.07

_B = 128
_M = 16384
_D = 2048
_BLK = 2048
_NBLK = _M // _BLK

_INT_MAX = 2147483647


def _loss_body(logits_ref, cid_ref, tid_ref, cam_ref, trk_ref, loss_ref,
               m_scr, s_scr, ps_scr, np_scr):
    j = pl.program_id(0)

    @pl.when(j == 0)
    def _init():
        m_scr[...] = jnp.full(m_scr.shape, -jnp.inf, m_scr.dtype)
        s_scr[...] = jnp.zeros(s_scr.shape, s_scr.dtype)
        ps_scr[...] = jnp.zeros(ps_scr.shape, ps_scr.dtype)
        np_scr[...] = jnp.zeros(np_scr.shape, np_scr.dtype)

    logits = logits_ref[...]
    cam = cid_ref[...] == cam_ref[...]
    pos = jnp.logical_and(cam, tid_ref[...] == trk_ref[...])

    a = logits * jnp.float32(1.0 / _TEMPERATURE)

    blk_max = jnp.max(jnp.where(cam, a, -jnp.inf), axis=1, keepdims=True)
    m_old = m_scr[...]
    m_new = jnp.maximum(m_old, blk_max)
    scale = jnp.where(m_old == m_new, jnp.float32(1.0), jnp.exp(m_old - m_new))
    blk_sum = jnp.sum(jnp.where(cam, jnp.exp(a - m_new), 0.0),
                      axis=1, keepdims=True)
    s_scr[...] = s_scr[...] * scale + blk_sum
    m_scr[...] = m_new

    ps_scr[...] = ps_scr[...] + jnp.sum(jnp.where(pos, a, 0.0),
                                        axis=1, keepdims=True)
    np_scr[...] = np_scr[...] + jnp.sum(jnp.where(pos, 1.0, 0.0),
                                        axis=1, keepdims=True)

    @pl.when(j == _NBLK - 1)
    def _fin():
        mean_lp = ps_scr[...] / np_scr[...] - (m_scr[...] + jnp.log(s_scr[...]))
        loss_i = -(_TEMPERATURE / _BASE_TEMPERATURE) * mean_lp
        loss_ref[...] = jnp.sum(loss_i, axis=0, keepdims=True) * jnp.float32(1.0 / _B)


_loss_call = pl.pallas_call(
    _loss_body,
    grid=(_NBLK,),
    in_specs=[
        pl.BlockSpec((_B, _BLK), lambda j: (0, j)),
        pl.BlockSpec((1, _BLK), lambda j: (0, j)),
        pl.BlockSpec((1, _BLK), lambda j: (0, j)),
        pl.BlockSpec((_B, 1), lambda j: (0, 0)),
        pl.BlockSpec((_B, 1), lambda j: (0, 0)),
    ],
    out_specs=pl.BlockSpec((1, 1), lambda j: (0, 0)),
    out_shape=jax.ShapeDtypeStruct((1, 1), jnp.float32),
    scratch_shapes=[
        pltpu.VMEM((_B, 1), jnp.float32),
        pltpu.VMEM((_B, 1), jnp.float32),
        pltpu.VMEM((_B, 1), jnp.float32),
        pltpu.VMEM((_B, 1), jnp.float32),
    ],
)


_NC = 2
_NS = 16
_NW = _NC * _NS
_RPW = _B // _NW
_L = 16
_NCHUNK = _M // _L


def _hard_body(logits_hbm, cid_hbm, tid_hbm, cam_hbm, trk_hbm, mem_hbm,
               out_hbm, lg_v, cid_v, tid_v, cam_v, trk_v, rows_v, sem):
    wid = lax.axis_index("s") * _NC + lax.axis_index("c")
    base = wid * _RPW

    copies = [
        pltpu.async_copy(cid_hbm, cid_v, sem),
        pltpu.async_copy(tid_hbm, tid_v, sem),
        pltpu.async_copy(cam_hbm, cam_v.at[pl.ds(0, _B)], sem),
        pltpu.async_copy(trk_hbm, trk_v.at[pl.ds(0, _B)], sem),
    ]
    for r in range(_RPW):
        copies.append(pltpu.async_copy(
            logits_hbm.at[base + r], lg_v.at[pl.ds(r * _M, _M)], sem))
    for c in copies:
        c.wait()

    lanes = lax.broadcasted_iota(jnp.int32, (_L,), 0)
    camv = cam_v[pl.ds(base, _L)]
    trkv = trk_v[pl.ds(base, _L)]

    row_dmas = []
    for r in range(_RPW):
        cam_s = camv[r]
        trk_s = trkv[r]

        def chunk(c, carry, r=r, cam_s=cam_s, trk_s=trk_s):
            vmin, vidx, pbase = carry
            off = c * _L
            lg = lg_v[pl.ds(r * _M + off, _L)]
            cid = cid_v[pl.ds(off, _L)]
            tid = tid_v[pl.ds(off, _L)]
            pos = jnp.logical_and(cid == cam_s, tid == trk_s)
            take = jnp.logical_and(pos, lg < vmin)
            vmin = jnp.where(take, lg, vmin)
            vidx = jnp.where(take, pbase, vidx)
            return vmin, vidx, pbase + _L

        vmin, vidx, _ = lax.fori_loop(
            0, _NCHUNK, chunk,
            (jnp.full((_L,), jnp.inf, jnp.float32),
             jnp.zeros((_L,), jnp.int32), lanes))

        m = vmin[0]
        g = vidx[0]
        for j in range(1, _L):
            v = vmin[j]
            i = vidx[j]
            better = v < m
            tie = jnp.logical_and(v == m, i < g)
            m = jnp.where(better, v, m)
            g = jnp.where(jnp.logical_or(better, tie), i, g)

        row_dmas.append(pltpu.async_copy(mem_hbm.at[g], rows_v.at[r], sem))

    for dma in row_dmas:
        dma.wait()
    pltpu.sync_copy(rows_v, out_hbm.at[pl.ds(base, _RPW)])


_hard_call = pl.kernel(
    _hard_body,
    out_type=jax.ShapeDtypeStruct((_B, _D), jnp.float32),
    mesh=plsc.VectorSubcoreMesh(core_axis_name="c", subcore_axis_name="s"),
    scratch_types=[
        pltpu.VMEM((_RPW * _M,), jnp.float32),
        pltpu.VMEM((_M,), jnp.int32),
        pltpu.VMEM((_M,), jnp.int32),
        pltpu.VMEM((_B + _L,), jnp.int32),
        pltpu.VMEM((_B + _L,), jnp.int32),
        pltpu.VMEM((_RPW, _D), jnp.float32),
        pltpu.SemaphoreType.DMA,
    ],
)


def kernel(mem, logits, mem_CID, mem_TID, camids, trackids):
    loss2 = _loss_call(
        logits,
        mem_CID.reshape(1, _M),
        mem_TID.reshape(1, _M),
        camids.reshape(_B, 1),
        trackids.reshape(_B, 1),
    )
    hard_pos = _hard_call(logits, mem_CID, mem_TID, camids, trackids, mem)
    return loss2[0, 0], hard_pos


# speedup vs baseline: 1.2883x; 1.2883x over previous
import jax
import jax.numpy as jnp
from jax import lax
from jax.experimental import pallas as pl
from jax.experimental.pallas import tpu as pltpu
from jax.experimental.pallas import tpu_sc as plsc

_TEMPERATURE = 0.07
_BASE_TEMPERATURE = 0.07

_B = 128
_M = 16384
_D = 2048
_BLK = 2048
_NBLK = _M // _BLK

_INT_MAX = 2147483647


def _loss_body(logits_ref, cid_ref, tid_ref, cam_ref, trk_ref, loss_ref,
               m_scr, s_scr, ps_scr, np_scr):
    j = pl.program_id(0)

    @pl.when(j == 0)
    def _init():
        m_scr[...] = jnp.full(m_scr.shape, -jnp.inf, m_scr.dtype)
        s_scr[...] = jnp.zeros(s_scr.shape, s_scr.dtype)
        ps_scr[...] = jnp.zeros(ps_scr.shape, ps_scr.dtype)
        np_scr[...] = jnp.zeros(np_scr.shape, np_scr.dtype)

    logits = logits_ref[...]
    cam = cid_ref[...] == cam_ref[...]
    pos = jnp.logical_and(cam, tid_ref[...] == trk_ref[...])

    a = logits * jnp.float32(1.0 / _TEMPERATURE)

    blk_max = jnp.max(jnp.where(cam, a, -jnp.inf), axis=1, keepdims=True)
    m_old = m_scr[...]
    m_new = jnp.maximum(m_old, blk_max)
    scale = jnp.where(m_old == m_new, jnp.float32(1.0), jnp.exp(m_old - m_new))
    blk_sum = jnp.sum(jnp.where(cam, jnp.exp(a - m_new), 0.0),
                      axis=1, keepdims=True)
    s_scr[...] = s_scr[...] * scale + blk_sum
    m_scr[...] = m_new

    ps_scr[...] = ps_scr[...] + jnp.sum(jnp.where(pos, a, 0.0),
                                        axis=1, keepdims=True)
    np_scr[...] = np_scr[...] + jnp.sum(jnp.where(pos, 1.0, 0.0),
                                        axis=1, keepdims=True)

    @pl.when(j == _NBLK - 1)
    def _fin():
        mean_lp = ps_scr[...] / np_scr[...] - (m_scr[...] + jnp.log(s_scr[...]))
        loss_i = -(_TEMPERATURE / _BASE_TEMPERATURE) * mean_lp
        loss_ref[...] = jnp.sum(loss_i, axis=0, keepdims=True) * jnp.float32(1.0 / _B)


_loss_call = pl.pallas_call(
    _loss_body,
    grid=(_NBLK,),
    in_specs=[
        pl.BlockSpec((_B, _BLK), lambda j: (0, j)),
        pl.BlockSpec((1, _BLK), lambda j: (0, j)),
        pl.BlockSpec((1, _BLK), lambda j: (0, j)),
        pl.BlockSpec((_B, 1), lambda j: (0, 0)),
        pl.BlockSpec((_B, 1), lambda j: (0, 0)),
    ],
    out_specs=pl.BlockSpec((1, 1), lambda j: (0, 0)),
    out_shape=jax.ShapeDtypeStruct((1, 1), jnp.float32),
    scratch_shapes=[
        pltpu.VMEM((_B, 1), jnp.float32),
        pltpu.VMEM((_B, 1), jnp.float32),
        pltpu.VMEM((_B, 1), jnp.float32),
        pltpu.VMEM((_B, 1), jnp.float32),
    ],
)


_NC = 2
_NS = 16
_NW = _NC * _NS
_RPW = _B // _NW
_L = 16
_NCHUNK = _M // _L


def _hard_body(logits_hbm, cid_hbm, tid_hbm, cam_hbm, trk_hbm, mem_hbm,
               out_hbm, lg_v, cid_v, tid_v, cam_v, trk_v, rows_v, sem):
    wid = lax.axis_index("s") * _NC + lax.axis_index("c")
    base = wid * _RPW

    copies = [
        pltpu.async_copy(cid_hbm, cid_v, sem),
        pltpu.async_copy(tid_hbm, tid_v, sem),
        pltpu.async_copy(cam_hbm, cam_v.at[pl.ds(0, _B)], sem),
        pltpu.async_copy(trk_hbm, trk_v.at[pl.ds(0, _B)], sem),
    ]
    for r in range(_RPW):
        copies.append(pltpu.async_copy(
            logits_hbm.at[base + r], lg_v.at[pl.ds(r * _M, _M)], sem))
    for c in copies:
        c.wait()

    lanes = lax.broadcasted_iota(jnp.int32, (_L,), 0)
    camv = cam_v[pl.ds(base, _L)]
    trkv = trk_v[pl.ds(base, _L)]
    keya = [(camv[r] << 16) | trkv[r] for r in range(_RPW)]

    def chunk(c, carry):
        state, pbase = carry
        off = c * _L
        cid = cid_v[pl.ds(off, _L)]
        tid = tid_v[pl.ds(off, _L)]
        key = (cid << 16) | tid
        new_state = []
        for r in range(_RPW):
            vmin, vidx = state[2 * r], state[2 * r + 1]
            lg = lg_v[pl.ds(r * _M + off, _L)]
            take = jnp.logical_and(key == keya[r], lg < vmin)
            new_state.append(jnp.where(take, lg, vmin))
            new_state.append(jnp.where(take, pbase, vidx))
        return tuple(new_state), pbase + _L

    init = []
    for r in range(_RPW):
        init.append(jnp.full((_L,), jnp.inf, jnp.float32))
        init.append(jnp.zeros((_L,), jnp.int32))
    state, _ = lax.fori_loop(0, _NCHUNK, chunk, (tuple(init), lanes))

    row_dmas = []
    for r in range(_RPW):
        vmin, vidx = state[2 * r], state[2 * r + 1]
        m = vmin[0]
        g = vidx[0]
        for j in range(1, _L):
            v = vmin[j]
            i = vidx[j]
            better = v < m
            tie = jnp.logical_and(v == m, i < g)
            m = jnp.where(better, v, m)
            g = jnp.where(jnp.logical_or(better, tie), i, g)
        row_dmas.append(pltpu.async_copy(mem_hbm.at[g], rows_v.at[r], sem))

    for dma in row_dmas:
        dma.wait()
    pltpu.sync_copy(rows_v, out_hbm.at[pl.ds(base, _RPW)])


_hard_call = pl.kernel(
    _hard_body,
    out_type=jax.ShapeDtypeStruct((_B, _D), jnp.float32),
    mesh=plsc.VectorSubcoreMesh(core_axis_name="c", subcore_axis_name="s"),
    scratch_types=[
        pltpu.VMEM((_RPW * _M,), jnp.float32),
        pltpu.VMEM((_M,), jnp.int32),
        pltpu.VMEM((_M,), jnp.int32),
        pltpu.VMEM((_B + _L,), jnp.int32),
        pltpu.VMEM((_B + _L,), jnp.int32),
        pltpu.VMEM((_RPW, _D), jnp.float32),
        pltpu.SemaphoreType.DMA,
    ],
)


def kernel(mem, logits, mem_CID, mem_TID, camids, trackids):
    loss2 = _loss_call(
        logits,
        mem_CID.reshape(1, _M),
        mem_TID.reshape(1, _M),
        camids.reshape(_B, 1),
        trackids.reshape(_B, 1),
    )
    hard_pos = _hard_call(logits, mem_CID, mem_TID, camids, trackids, mem)
    return loss2[0, 0], hard_pos
